# initial kernel scaffold (unmeasured)
import jax
import jax.numpy as jnp
from jax import lax
from jax.experimental import pallas as pl
from jax.experimental.pallas import tpu as pltpu

N_DEV = 32
M_BLK = 128
K = 4096
N = 8192
N_TILE = 1024
N_TILES = N // N_TILE


def kernel(x, w_mat, scale_x, scale_w):
    def body(x_ref, w_ref, sx_ref, sw_ref, out_ref, xt_ref, send_sems, recv_sems):
        t = pl.program_id(0)
        my = lax.axis_index("i")

        @pl.when(t == 0)
        def _():
            xt_ref[:, pl.ds(my * M_BLK, M_BLK)] = x_ref[pl.ds(my * M_BLK, M_BLK), :]

            for off in range(1, N_DEV):
                d = lax.rem(my + off, N_DEV)
                send = pltpu.make_async_remote_copy(
                    src_ref=x_ref.at[pl.ds(d * M_BLK, M_BLK), :],
                    dst_ref=xt_ref.at[:, pl.ds(my * M_BLK, M_BLK)],
                    send_sem=send_sems.at[off],
                    recv_sem=recv_sems.at[my],
                    device_id=(d,),
                    device_id_type=pl.DeviceIdType.MESH,
                )
                send.start()

            for off in range(1, N_DEV):
                j = lax.rem(my + off, N_DEV)
                recv = pltpu.make_async_remote_copy(
                    src_ref=x_ref.at[pl.ds(j * M_BLK, M_BLK), :],
                    dst_ref=xt_ref.at[:, pl.ds(j * M_BLK, M_BLK)],
                    send_sem=send_sems.at[off],
                    recv_sem=recv_sems.at[j],
                    device_id=(j,),
                    device_id_type=pl.DeviceIdType.MESH,
                )
                recv.wait_recv()

            for off in range(1, N_DEV):
                d = lax.rem(my + off, N_DEV)
                send = pltpu.make_async_remote_copy(
                    src_ref=x_ref.at[pl.ds(d * M_BLK, M_BLK), :],
                    dst_ref=xt_ref.at[:, pl.ds(my * M_BLK, M_BLK)],
                    send_sem=send_sems.at[off],
                    recv_sem=recv_sems.at[my],
                    device_id=(d,),
                    device_id_type=pl.DeviceIdType.MESH,
                )
                send.wait_send()

        acc = lax.dot_general(
            xt_ref[...],
            w_ref[...],
            dimension_numbers=(((1,), (0,)), ((), ())),
            preferred_element_type=jnp.float32,
        )
        y = acc * (sx_ref[0] * sw_ref[0])
        out_ref[...] = y * (1.0 / (1.0 + jnp.exp(-jnp.clip(y, -60.0, 60.0))))

    return pl.pallas_call(
        body,
        grid=(N_TILES,),
        in_specs=[
            pl.BlockSpec((K, M_BLK), lambda t: (0, 0)),
            pl.BlockSpec((K, N_TILE), lambda t: (0, t)),
            pl.BlockSpec(memory_space=pltpu.SMEM),
            pl.BlockSpec(memory_space=pltpu.SMEM),
        ],
        out_specs=pl.BlockSpec((M_BLK, N_TILE), lambda t: (0, t)),
        out_shape=jax.ShapeDtypeStruct((M_BLK, N), jnp.float32),
        scratch_shapes=[
            pltpu.VMEM((M_BLK, K), x.dtype),
            pltpu.SemaphoreType.DMA((N_DEV,)),
            pltpu.SemaphoreType.DMA((N_DEV,)),
        ],
        compiler_params=pltpu.CompilerParams(
            dimension_semantics=("arbitrary",),
        ),
    )(x, w_mat, scale_x, scale_w)


# baseline (device time: 64236 ns/iter reference)
import jax
import jax.numpy as jnp
from jax import lax
from jax.experimental import pallas as pl
from jax.experimental.pallas import tpu as pltpu

N_DEV = 32
M_BLK = 128
K = 4096
N = 8192
N_TILE = 512
N_TILES = N // N_TILE

FP8 = jnp.float8_e5m2


def kernel(x, w_mat, scale_x, scale_w):
    def body(x_ref, w_ref, sx_ref, sw_ref, out_ref,
             xs_ref, xt_ref, send_sems, recv_sems):
        t = pl.program_id(0)
        my = lax.axis_index("i")

        @pl.when(t == 0)
        def _():
            xs_ref[...] = x_ref[...].astype(FP8)
            xt_ref[:, pl.ds(my * M_BLK, M_BLK)] = xs_ref[pl.ds(my * M_BLK, M_BLK), :]

            for off in range(1, N_DEV):
                d = lax.rem(my + off, N_DEV)
                send = pltpu.make_async_remote_copy(
                    src_ref=xs_ref.at[pl.ds(d * M_BLK, M_BLK), :],
                    dst_ref=xt_ref.at[:, pl.ds(my * M_BLK, M_BLK)],
                    send_sem=send_sems.at[off],
                    recv_sem=recv_sems.at[my],
                    device_id=(d,),
                    device_id_type=pl.DeviceIdType.MESH,
                )
                send.start()

            for off in range(1, N_DEV):
                j = lax.rem(my + off, N_DEV)
                recv = pltpu.make_async_remote_copy(
                    src_ref=xs_ref.at[pl.ds(j * M_BLK, M_BLK), :],
                    dst_ref=xt_ref.at[:, pl.ds(j * M_BLK, M_BLK)],
                    send_sem=send_sems.at[off],
                    recv_sem=recv_sems.at[j],
                    device_id=(j,),
                    device_id_type=pl.DeviceIdType.MESH,
                )
                recv.wait_recv()

            for off in range(1, N_DEV):
                d = lax.rem(my + off, N_DEV)
                send = pltpu.make_async_remote_copy(
                    src_ref=xs_ref.at[pl.ds(d * M_BLK, M_BLK), :],
                    dst_ref=xt_ref.at[:, pl.ds(my * M_BLK, M_BLK)],
                    send_sem=send_sems.at[off],
                    recv_sem=recv_sems.at[my],
                    device_id=(d,),
                    device_id_type=pl.DeviceIdType.MESH,
                )
                send.wait_send()

        wq = w_ref[...].astype(FP8)
        acc = lax.dot_general(
            xt_ref[...],
            wq,
            dimension_numbers=(((1,), (0,)), ((), ())),
            preferred_element_type=jnp.float32,
        )
        y = acc * (sx_ref[0] * sw_ref[0])
        out_ref[...] = y * (1.0 / (1.0 + jnp.exp(-jnp.clip(y, -60.0, 60.0))))

    return pl.pallas_call(
        body,
        grid=(N_TILES,),
        in_specs=[
            pl.BlockSpec((K, M_BLK), lambda t: (0, 0)),
            pl.BlockSpec((K, N_TILE), lambda t: (0, t)),
            pl.BlockSpec(memory_space=pltpu.SMEM),
            pl.BlockSpec(memory_space=pltpu.SMEM),
        ],
        out_specs=pl.BlockSpec((M_BLK, N_TILE), lambda t: (0, t)),
        out_shape=jax.ShapeDtypeStruct((M_BLK, N), jnp.float32),
        scratch_shapes=[
            pltpu.VMEM((K, M_BLK), FP8),
            pltpu.VMEM((M_BLK, K), FP8),
            pltpu.SemaphoreType.DMA((N_DEV,)),
            pltpu.SemaphoreType.DMA((N_DEV,)),
        ],
        compiler_params=pltpu.CompilerParams(
            dimension_semantics=("arbitrary",),
        ),
    )(x, w_mat, scale_x, scale_w)
